# baseline (speedup 1.0000x reference)
import jax
import jax.numpy as jnp
from jax import lax
from jax.experimental import pallas as pl
from jax.experimental.pallas import tpu as pltpu
from jax.experimental.pallas import tpu_sc as plsc

B, N, K, C, D = 2, 10000, 16, 8, 64
UPDATE_RATE = 0.01
MARGIN = 0.5

NC, NS = 2, 16
NW = NC * NS
TOT = B * N
CHUNK = 8
NCH = TOT // CHUNK
NCH_B = N // CHUNK
QPW = -(-NCH // NW)
QMAX = (QPW + 2) // 3

NB_A = 2000


def _stage_a_body(f_ref, x_ref, cent_ref, w1_ref, b1_ref, w2_ref, b2_ref,
                  bn_ref, z_ref):
    f0 = f_ref[0]
    f1 = f_ref[1]
    inv_bd = 1.0 / (B * D)
    mean = (jnp.sum(f0, axis=1, keepdims=True)
            + jnp.sum(f1, axis=1, keepdims=True)) * inv_bd
    d0 = f0 - mean
    d1 = f1 - mean
    var = (jnp.sum(d0 * d0, axis=1, keepdims=True)
           + jnp.sum(d1 * d1, axis=1, keepdims=True)) * inv_bd
    scale = bn_ref[:, 0:1] * lax.rsqrt(var + 1e-5)
    bias = bn_ref[:, 1:2]

    w1a = w1_ref[0:D, :]
    w1b = w1_ref[D:2 * D, :]
    cpart = jnp.dot(cent_ref[...], w1b,
                    preferred_element_type=jnp.float32) + b1_ref[...]
    cp8 = jnp.concatenate([cpart[c:c + 1, :] for c in range(C)], axis=1)
    w2rep = jnp.concatenate([w2_ref[...]] * C, axis=0)
    rr = lax.broadcasted_iota(jnp.int32, (C * D, C), 0)
    cc = lax.broadcasted_iota(jnp.int32, (C * D, C), 1)
    w2blk = jnp.where(rr // D == cc, w2rep, 0.0)
    b2 = b2_ref[...]

    for b, db in ((0, d0), (1, d1)):
        ffn = db * scale + bias
        xp = jnp.dot(ffn, w1a, preferred_element_type=jnp.float32)
        xp8 = jnp.concatenate([xp] * C, axis=1)
        h = jnp.maximum(xp8 + cp8, 0.0)
        sg = jnp.dot(h, w2blk, preferred_element_type=jnp.float32)
        s = jnp.maximum(sg + b2, 0.0)
        m = jnp.max(s, axis=1, keepdims=True)
        e = jnp.exp(s - m)
        simi = e / jnp.sum(e, axis=1, keepdims=True)
        sb = lax.bitcast_convert_type(
            (simi * (1.0 / K)).astype(jnp.bfloat16), jnp.uint16)
        sw = sb.astype(jnp.uint32)
        sw = sw | (sw << 16)
        spk = lax.bitcast_convert_type(sw, jnp.float32)
        pad = jnp.zeros((simi.shape[0], 128 - 32 - C), jnp.float32)
        z_ref[b] = jnp.concatenate([x_ref[b], spk, pad], axis=1)


def _stage_a(fushed, xpk, centroids, W1, b1r, W2, b2r, bnpack):
    grid = (N // NB_A,)
    return pl.pallas_call(
        _stage_a_body,
        grid=grid,
        in_specs=[
            pl.BlockSpec((B, NB_A, D), lambda i: (0, i, 0)),
            pl.BlockSpec((B, NB_A, 32), lambda i: (0, i, 0)),
            pl.BlockSpec((C, D), lambda i: (0, 0)),
            pl.BlockSpec((2 * D, D), lambda i: (0, 0)),
            pl.BlockSpec((1, D), lambda i: (0, 0)),
            pl.BlockSpec((D, 1), lambda i: (0, 0)),
            pl.BlockSpec((1, 1), lambda i: (0, 0)),
            pl.BlockSpec((NB_A, 2), lambda i: (i, 0)),
        ],
        out_specs=pl.BlockSpec((B, NB_A, 128), lambda i: (0, i, 0)),
        out_shape=jax.ShapeDtypeStruct((B, N, 128), jnp.float32),
    )(fushed, xpk, centroids, W1, b1r, W2, b2r, bnpack)


def _stage_b_body(z_hbm, idx_hbm, out_hbm, psum_hbm,
                  idx0, idx1, idx2, zr0, zr1, zr2, out0, out1, out2,
                  psum_v, psf_v,
                  gsem0, gsem1, gsem2, isem0, isem1, isem2,
                  osem0, osem1, osem2):
    wid = lax.axis_index("s") * NC + lax.axis_index("c")
    idx_v = (idx0, idx1, idx2)
    zr = (zr0, zr1, zr2)
    out_v = (out0, out1, out2)
    gsem = (gsem0, gsem1, gsem2)
    isem = (isem0, isem1, isem2)
    osem = (osem0, osem1, osem2)
    zero_bf = jnp.zeros((32,), jnp.bfloat16)
    zero16 = jnp.zeros((16,), jnp.float32)
    for c in range(C):
        for h in range(2):
            psum_v[c, pl.ds(16 * h, 16)] = zero16

    def out_slice(cid):
        bq = cid // NCH_B
        n0 = (cid - bq * NCH_B) * CHUNK
        return out_hbm.at[bq, pl.ds(n0, CHUNK)]

    for p in (0, 1, 2):
        cid0 = p * NW + wid
        pltpu.sync_copy(idx_hbm.at[pl.ds(cid0 * CHUNK * K, CHUNK * K)],
                        idx_v[p])
        pltpu.async_copy(z_hbm.at[idx_v[p]], zr[p], gsem[p])

    def tri_body(tt, _):
        for p in (0, 1, 2):
            q = 3 * tt + p
            cid = q * NW + wid
            cid2 = cid + 3 * NW

            @pl.when(cid < NCH)
            def _():
                pltpu.make_async_copy(z_hbm.at[idx_v[p]], zr[p],
                                      gsem[p]).wait()

                @pl.when(cid2 < NCH)
                def _():
                    pltpu.async_copy(
                        idx_hbm.at[pl.ds(cid2 * CHUNK * K, CHUNK * K)],
                        idx_v[p], isem[p])

                @pl.when(cid >= 3 * NW)
                def _():
                    pltpu.make_async_copy(out_v[p], out_slice(cid),
                                          osem[p]).wait()

                def node_body(mm, psums):
                    accs = [[[zero_bf, zero_bf] for _ in range(C)]
                            for _ in range(2)]
                    for k in range(K):
                        for mo in (0, 1):
                            row = (2 * mm + mo) * K + k
                            xh = [plsc.bitcast(
                                zr[p][row, pl.ds(16 * h, 16)],
                                jnp.bfloat16) for h in range(2)]
                            srow = zr[p][row, pl.ds(32, 16)]
                            for c in range(C):
                                sb = plsc.bitcast(
                                    lax.broadcast_in_dim(srow[c], (16,), ()),
                                    jnp.bfloat16)
                                accs[mo][c][0] = accs[mo][c][0] + sb * xh[0]
                                accs[mo][c][1] = accs[mo][c][1] + sb * xh[1]
                    new_psums = list(psums)
                    for mo in (0, 1):
                        for c in range(C):
                            for h in range(2):
                                a, b = plsc.unpack(
                                    accs[mo][c][h],
                                    format=plsc.PackFormat.INTERLEAVED)
                                out_v[p][2 * mm + mo, c, pl.ds(32 * h, 16)] = a
                                out_v[p][2 * mm + mo, c,
                                         pl.ds(32 * h + 16, 16)] = b
                                new_psums[2 * c + h] = (
                                    new_psums[2 * c + h] + accs[mo][c][h])
                    return tuple(new_psums)

                psums = lax.fori_loop(0, CHUNK // 2, node_body,
                                      (zero_bf,) * (2 * C))
                for c in range(C):
                    for h in range(2):
                        pb = plsc.bitcast(psum_v[c, pl.ds(16 * h, 16)],
                                          jnp.bfloat16) + psums[2 * c + h]
                        psum_v[c, pl.ds(16 * h, 16)] = plsc.bitcast(
                            pb, jnp.float32)
                pltpu.async_copy(out_v[p], out_slice(cid), osem[p])

                @pl.when(cid2 < NCH)
                def _():
                    pltpu.make_async_copy(
                        idx_hbm.at[pl.ds(cid2 * CHUNK * K, CHUNK * K)],
                        idx_v[p], isem[p]).wait()
                    pltpu.async_copy(z_hbm.at[idx_v[p]], zr[p], gsem[p])

        return ()

    lax.fori_loop(0, QMAX, tri_body, ())
    nq = (NCH - wid + NW - 1) // NW
    for p in (0, 1, 2):
        qlast = nq - 1 - ((nq - 1 - p) % 3)
        pltpu.make_async_copy(out_v[p], out_slice(qlast * NW + wid),
                              osem[p]).wait()
    for c in range(C):
        for h in range(2):
            a, b = plsc.unpack(
                plsc.bitcast(psum_v[c, pl.ds(16 * h, 16)], jnp.bfloat16),
                format=plsc.PackFormat.INTERLEAVED)
            psf_v[c, pl.ds(32 * h, 16)] = a
            psf_v[c, pl.ds(32 * h + 16, 16)] = b
    pltpu.sync_copy(psf_v, psum_hbm.at[wid])


def _stage_b(z_rows, idx_flat):
    mesh = plsc.VectorSubcoreMesh(core_axis_name="c", subcore_axis_name="s")
    run = pl.kernel(
        _stage_b_body,
        out_type=[
            jax.ShapeDtypeStruct((B, N, C, D), jnp.float32),
            jax.ShapeDtypeStruct((NW, C, D), jnp.float32),
        ],
        mesh=mesh,
        compiler_params=pltpu.CompilerParams(needs_layout_passes=False),
        scratch_types=[
            pltpu.VMEM((CHUNK * K,), jnp.int32),
            pltpu.VMEM((CHUNK * K,), jnp.int32),
            pltpu.VMEM((CHUNK * K,), jnp.int32),
            pltpu.VMEM((CHUNK * K, 128), jnp.float32),
            pltpu.VMEM((CHUNK * K, 128), jnp.float32),
            pltpu.VMEM((CHUNK * K, 128), jnp.float32),
            pltpu.VMEM((CHUNK, C, D), jnp.float32),
            pltpu.VMEM((CHUNK, C, D), jnp.float32),
            pltpu.VMEM((CHUNK, C, D), jnp.float32),
            pltpu.VMEM((C, 32), jnp.float32),
            pltpu.VMEM((C, D), jnp.float32),
        ] + [pltpu.SemaphoreType.DMA] * 9,
    )
    return run(z_rows, idx_flat)


def _stage_c_body(psum_ref, cent_ref, out_ref):
    acc = psum_ref[0]
    for w in range(1, NW):
        acc = acc + psum_ref[w]
    u = acc * (1.0 / TOT)
    nc = (1.0 - UPDATE_RATE) * cent_ref[...] + UPDATE_RATE * u

    adj = jnp.mean(nc, axis=0, keepdims=True)
    xc = nc - adj
    nsq = jnp.sum(xc * xc, axis=1, keepdims=True)
    ones = jnp.ones_like(nsq)
    x1_ = jnp.concatenate([-2.0 * xc, nsq, ones], axis=1)
    x2_ = jnp.concatenate([xc, ones, nsq], axis=1)
    res = lax.dot_general(x1_, x2_, (((1,), (1,)), ((), ())),
                          preferred_element_type=jnp.float32)
    dist = jnp.sqrt(jnp.clip(res, 1e-30, None))
    ii = lax.broadcasted_iota(jnp.int32, (C, C), 0)
    jj = lax.broadcasted_iota(jnp.int32, (C, C), 1)
    target = jnp.where(ii == jj, 0.0, MARGIN)
    l = jnp.maximum(target - dist, 0.0)
    out_ref[...] = jnp.reshape(jnp.sum(l * l), (1, 1))


def _stage_c(psum, centroids):
    return pl.pallas_call(
        _stage_c_body,
        out_shape=jax.ShapeDtypeStruct((1, 1), jnp.float32),
    )(psum, centroids)


@jax.jit
def kernel(fushed_features, input_data, adj_mx_topk_index, centroids,
           W1, b1, W2, b2, bn_weight, bn_bias):
    b1r = jnp.reshape(b1, (1, D))
    b2r = jnp.reshape(b2, (1, 1))
    bnpack = jnp.stack([bn_weight, bn_bias], axis=1)

    xb = jnp.reshape(input_data, (B, N, D)).astype(jnp.bfloat16)
    xu = lax.bitcast_convert_type(xb, jnp.uint16).astype(jnp.uint32)
    words = []
    for g in range(2):
        lo = xu[:, :, 32 * g:32 * g + 16]
        hi = xu[:, :, 32 * g + 16:32 * g + 32]
        words.append(lo | (hi << 16))
    xpk = lax.bitcast_convert_type(
        jnp.concatenate(words, axis=-1), jnp.float32)

    z = _stage_a(fushed_features, xpk, centroids,
                 W1, b1r, W2, b2r, bnpack)

    idx_flat = jnp.reshape(
        adj_mx_topk_index
        + (jnp.arange(B, dtype=jnp.int32) * N)[:, None, None],
        (TOT * K,))

    z_rows = jnp.reshape(z, (TOT, 128))
    updated_input, psum = _stage_b(z_rows, idx_flat)

    loss = _stage_c(psum, centroids)
    return updated_input, jnp.reshape(loss, ())


# speedup vs baseline: 1.2311x; 1.2311x over previous
import jax
import jax.numpy as jnp
from jax import lax
from jax.experimental import pallas as pl
from jax.experimental.pallas import tpu as pltpu
from jax.experimental.pallas import tpu_sc as plsc

B, N, K, C, D = 2, 10000, 16, 8, 64
UPDATE_RATE = 0.01
MARGIN = 0.5

NC, NS = 2, 16
NW = NC * NS
TOT = B * N
CHUNK = 8
NCH = TOT // CHUNK
NCH_B = N // CHUNK
QPW = -(-NCH // NW)
QMAX = (QPW + 2) // 3

NB_A = 1280


def _stage_a_body(f_ref, x_ref, cent_ref, w1_ref, b1_ref, w2_ref, b2_ref,
                  bn_ref, z_ref):
    f0 = f_ref[0]
    f1 = f_ref[1]
    inv_bd = 1.0 / (B * D)
    mean = (jnp.sum(f0, axis=0, keepdims=True)
            + jnp.sum(f1, axis=0, keepdims=True)) * inv_bd
    d0 = f0 - mean
    d1 = f1 - mean
    var = (jnp.sum(d0 * d0, axis=0, keepdims=True)
           + jnp.sum(d1 * d1, axis=0, keepdims=True)) * inv_bd
    scale = bn_ref[0:1, :] * lax.rsqrt(var + 1e-5)
    bias = bn_ref[1:2, :]

    w1a = w1_ref[0:D, :]
    w1b = w1_ref[D:2 * D, :]
    cpart_t = lax.dot_general(w1b, cent_ref[...],
                              (((0,), (1,)), ((), ())),
                              preferred_element_type=jnp.float32)
    cpart_t = cpart_t + b1_ref[...]
    w2rep = jnp.concatenate([w2_ref[...]] * C, axis=0)
    rr = lax.broadcasted_iota(jnp.int32, (C * D, C), 0)
    cc = lax.broadcasted_iota(jnp.int32, (C * D, C), 1)
    w2blk = jnp.where(rr // D == cc, w2rep, 0.0)
    b2 = b2_ref[...]

    for b, db, xb in ((0, d0, x_ref[0]), (1, d1, x_ref[1])):
        ffn = db * scale + bias
        xp = lax.dot_general(w1a, ffn, (((0,), (0,)), ((), ())),
                             preferred_element_type=jnp.float32)
        h8 = jnp.concatenate(
            [jnp.maximum(xp + cpart_t[:, c:c + 1], 0.0) for c in range(C)],
            axis=0)
        sg = lax.dot_general(w2blk, h8, (((0,), (0,)), ((), ())),
                             preferred_element_type=jnp.float32)
        st = jnp.maximum(sg + b2, 0.0)
        m = jnp.max(st, axis=0, keepdims=True)
        e = jnp.exp(st - m)
        simi = e / jnp.sum(e, axis=0, keepdims=True)
        sb = lax.bitcast_convert_type(
            (simi * (1.0 / K)).astype(jnp.bfloat16), jnp.uint16)
        sw = sb.astype(jnp.uint32)
        sw = sw | (sw << 16)
        spk = lax.bitcast_convert_type(sw, jnp.float32)
        xu = lax.bitcast_convert_type(xb.astype(jnp.bfloat16),
                                      jnp.uint16).astype(jnp.uint32)
        lo = jnp.concatenate([xu[0:16], xu[32:48]], axis=0)
        hi = jnp.concatenate([xu[16:32], xu[48:64]], axis=0)
        wpk = lax.bitcast_convert_type(lo | (hi << 16), jnp.float32)
        pad = jnp.zeros((z_ref.shape[1], 128 - 32 - C), jnp.float32)
        z_ref[b] = jnp.concatenate(
            [jnp.transpose(wpk), jnp.transpose(spk), pad], axis=1)


def _stage_a(fushed_t, x_t, centroids, W1, b1c, W2, b2r, bnpack):
    grid = (-(-N // NB_A),)
    return pl.pallas_call(
        _stage_a_body,
        grid=grid,
        in_specs=[
            pl.BlockSpec((B, D, NB_A), lambda i: (0, 0, i)),
            pl.BlockSpec((B, D, NB_A), lambda i: (0, 0, i)),
            pl.BlockSpec((C, D), lambda i: (0, 0)),
            pl.BlockSpec((2 * D, D), lambda i: (0, 0)),
            pl.BlockSpec((D, 1), lambda i: (0, 0)),
            pl.BlockSpec((D, 1), lambda i: (0, 0)),
            pl.BlockSpec((1, 1), lambda i: (0, 0)),
            pl.BlockSpec((2, NB_A), lambda i: (0, i)),
        ],
        out_specs=pl.BlockSpec((B, NB_A, 128), lambda i: (0, i, 0)),
        out_shape=jax.ShapeDtypeStruct((B, N, 128), jnp.float32),
    )(fushed_t, x_t, centroids, W1, b1c, W2, b2r, bnpack)


def _stage_b_body(z_hbm, idx_hbm, out_hbm, psum_hbm,
                  idx0, idx1, idx2, zr0, zr1, zr2, out0, out1, out2,
                  psum_v, psf_v,
                  gsem0, gsem1, gsem2, isem0, isem1, isem2,
                  osem0, osem1, osem2):
    wid = lax.axis_index("s") * NC + lax.axis_index("c")
    idx_v = (idx0, idx1, idx2)
    zr = (zr0, zr1, zr2)
    out_v = (out0, out1, out2)
    gsem = (gsem0, gsem1, gsem2)
    isem = (isem0, isem1, isem2)
    osem = (osem0, osem1, osem2)
    zero_bf = jnp.zeros((32,), jnp.bfloat16)
    zero16 = jnp.zeros((16,), jnp.float32)
    for c in range(C):
        for h in range(2):
            psum_v[c, pl.ds(16 * h, 16)] = zero16

    def out_slice(cid):
        bq = cid // NCH_B
        n0 = (cid - bq * NCH_B) * CHUNK
        return out_hbm.at[bq, pl.ds(n0, CHUNK)]

    for p in (0, 1, 2):
        cid0 = p * NW + wid
        pltpu.sync_copy(idx_hbm.at[pl.ds(cid0 * CHUNK * K, CHUNK * K)],
                        idx_v[p])
        pltpu.async_copy(z_hbm.at[idx_v[p]], zr[p], gsem[p])

    def tri_body(tt, _):
        for p in (0, 1, 2):
            q = 3 * tt + p
            cid = q * NW + wid
            cid2 = cid + 3 * NW

            @pl.when(cid < NCH)
            def _():
                pltpu.make_async_copy(z_hbm.at[idx_v[p]], zr[p],
                                      gsem[p]).wait()

                @pl.when(cid2 < NCH)
                def _():
                    pltpu.async_copy(
                        idx_hbm.at[pl.ds(cid2 * CHUNK * K, CHUNK * K)],
                        idx_v[p], isem[p])

                @pl.when(cid >= 3 * NW)
                def _():
                    pltpu.make_async_copy(out_v[p], out_slice(cid),
                                          osem[p]).wait()

                def node_body(m, psums):
                    accs = [[zero_bf, zero_bf] for _ in range(C)]
                    for k in range(K):
                        row = m * K + k
                        xh = [plsc.bitcast(zr[p][row, pl.ds(16 * h, 16)],
                                           jnp.bfloat16) for h in range(2)]
                        srow = zr[p][row, pl.ds(32, 16)]
                        for c in range(C):
                            sb = plsc.bitcast(
                                lax.broadcast_in_dim(srow[c], (16,), ()),
                                jnp.bfloat16)
                            accs[c][0] = accs[c][0] + sb * xh[0]
                            accs[c][1] = accs[c][1] + sb * xh[1]
                    new_psums = []
                    for c in range(C):
                        for h in range(2):
                            a, b = plsc.unpack(
                                accs[c][h],
                                format=plsc.PackFormat.INTERLEAVED)
                            out_v[p][m, c, pl.ds(32 * h, 16)] = a
                            out_v[p][m, c, pl.ds(32 * h + 16, 16)] = b
                            new_psums.append(psums[2 * c + h] + accs[c][h])
                    return tuple(new_psums)

                psums = lax.fori_loop(0, CHUNK, node_body,
                                      (zero_bf,) * (2 * C))
                for c in range(C):
                    for h in range(2):
                        pb = plsc.bitcast(psum_v[c, pl.ds(16 * h, 16)],
                                          jnp.bfloat16) + psums[2 * c + h]
                        psum_v[c, pl.ds(16 * h, 16)] = plsc.bitcast(
                            pb, jnp.float32)
                pltpu.async_copy(out_v[p], out_slice(cid), osem[p])

                @pl.when(cid2 < NCH)
                def _():
                    pltpu.make_async_copy(
                        idx_hbm.at[pl.ds(cid2 * CHUNK * K, CHUNK * K)],
                        idx_v[p], isem[p]).wait()
                    pltpu.async_copy(z_hbm.at[idx_v[p]], zr[p], gsem[p])

        return ()

    lax.fori_loop(0, QMAX, tri_body, ())
    nq = (NCH - wid + NW - 1) // NW
    for p in (0, 1, 2):
        qlast = nq - 1 - ((nq - 1 - p) % 3)
        pltpu.make_async_copy(out_v[p], out_slice(qlast * NW + wid),
                              osem[p]).wait()
    for c in range(C):
        for h in range(2):
            a, b = plsc.unpack(
                plsc.bitcast(psum_v[c, pl.ds(16 * h, 16)], jnp.bfloat16),
                format=plsc.PackFormat.INTERLEAVED)
            psf_v[c, pl.ds(32 * h, 16)] = a
            psf_v[c, pl.ds(32 * h + 16, 16)] = b
    pltpu.sync_copy(psf_v, psum_hbm.at[wid])


def _stage_b(z_rows, idx_flat):
    mesh = plsc.VectorSubcoreMesh(core_axis_name="c", subcore_axis_name="s")
    run = pl.kernel(
        _stage_b_body,
        out_type=[
            jax.ShapeDtypeStruct((B, N, C, D), jnp.float32),
            jax.ShapeDtypeStruct((NW, C, D), jnp.float32),
        ],
        mesh=mesh,
        compiler_params=pltpu.CompilerParams(needs_layout_passes=False),
        scratch_types=[
            pltpu.VMEM((CHUNK * K,), jnp.int32),
            pltpu.VMEM((CHUNK * K,), jnp.int32),
            pltpu.VMEM((CHUNK * K,), jnp.int32),
            pltpu.VMEM((CHUNK * K, 128), jnp.float32),
            pltpu.VMEM((CHUNK * K, 128), jnp.float32),
            pltpu.VMEM((CHUNK * K, 128), jnp.float32),
            pltpu.VMEM((CHUNK, C, D), jnp.float32),
            pltpu.VMEM((CHUNK, C, D), jnp.float32),
            pltpu.VMEM((CHUNK, C, D), jnp.float32),
            pltpu.VMEM((C, 32), jnp.float32),
            pltpu.VMEM((C, D), jnp.float32),
        ] + [pltpu.SemaphoreType.DMA] * 9,
    )
    return run(z_rows, idx_flat)


def _stage_c_body(psum_ref, cent_ref, out_ref):
    acc = psum_ref[0]
    for w in range(1, NW):
        acc = acc + psum_ref[w]
    u = acc * (1.0 / TOT)
    nc = (1.0 - UPDATE_RATE) * cent_ref[...] + UPDATE_RATE * u

    adj = jnp.mean(nc, axis=0, keepdims=True)
    xc = nc - adj
    nsq = jnp.sum(xc * xc, axis=1, keepdims=True)
    ones = jnp.ones_like(nsq)
    x1_ = jnp.concatenate([-2.0 * xc, nsq, ones], axis=1)
    x2_ = jnp.concatenate([xc, ones, nsq], axis=1)
    res = lax.dot_general(x1_, x2_, (((1,), (1,)), ((), ())),
                          preferred_element_type=jnp.float32)
    dist = jnp.sqrt(jnp.clip(res, 1e-30, None))
    ii = lax.broadcasted_iota(jnp.int32, (C, C), 0)
    jj = lax.broadcasted_iota(jnp.int32, (C, C), 1)
    target = jnp.where(ii == jj, 0.0, MARGIN)
    l = jnp.maximum(target - dist, 0.0)
    out_ref[...] = jnp.reshape(jnp.sum(l * l), (1, 1))


def _stage_c(psum, centroids):
    return pl.pallas_call(
        _stage_c_body,
        out_shape=jax.ShapeDtypeStruct((1, 1), jnp.float32),
    )(psum, centroids)


@jax.jit
def kernel(fushed_features, input_data, adj_mx_topk_index, centroids,
           W1, b1, W2, b2, bn_weight, bn_bias):
    b1c = jnp.reshape(b1, (D, 1))
    b2r = jnp.reshape(b2, (1, 1))
    bnpack = jnp.stack([bn_weight, bn_bias], axis=0)

    fushed_t = jnp.transpose(fushed_features, (0, 2, 1))
    x_t = jnp.reshape(jnp.transpose(input_data, (0, 1, 3, 2)), (B, D, N))

    z = _stage_a(fushed_t, x_t, centroids,
                 W1, b1c, W2, b2r, bnpack)

    idx_flat = jnp.reshape(
        adj_mx_topk_index
        + (jnp.arange(B, dtype=jnp.int32) * N)[:, None, None],
        (TOT * K,))

    z_rows = jnp.reshape(z, (TOT, 128))
    updated_input, psum = _stage_b(z_rows, idx_flat)

    loss = _stage_c(psum, centroids)
    return updated_input, jnp.reshape(loss, ())


# speedup vs baseline: 1.2772x; 1.0374x over previous
import jax
import jax.numpy as jnp
from jax import lax
from jax.experimental import pallas as pl
from jax.experimental.pallas import tpu as pltpu
from jax.experimental.pallas import tpu_sc as plsc

B, N, K, C, D = 2, 10000, 16, 8, 64
UPDATE_RATE = 0.01
MARGIN = 0.5

NC, NS = 2, 16
NW = NC * NS
TOT = B * N
CHUNK = 8
NCH = TOT // CHUNK
NCH_B = N // CHUNK
QPW = -(-NCH // NW)
QMAX = (QPW + 2) // 3

NB_A = 1280


def _stage_a_body(f_ref, x_ref, cent_ref, w1_ref, b1_ref, w2_ref, b2_ref,
                  bn_ref, z_ref):
    f0 = f_ref[0]
    f1 = f_ref[1]
    inv_bd = 1.0 / (B * D)
    mean = (jnp.sum(f0, axis=0, keepdims=True)
            + jnp.sum(f1, axis=0, keepdims=True)) * inv_bd
    d0 = f0 - mean
    d1 = f1 - mean
    var = (jnp.sum(d0 * d0, axis=0, keepdims=True)
           + jnp.sum(d1 * d1, axis=0, keepdims=True)) * inv_bd
    scale = bn_ref[0:1, :] * lax.rsqrt(var + 1e-5)
    bias = bn_ref[1:2, :]

    w1a = w1_ref[0:D, :]
    w1b = w1_ref[D:2 * D, :]
    cpart_t = lax.dot_general(w1b, cent_ref[...],
                              (((0,), (1,)), ((), ())),
                              preferred_element_type=jnp.float32)
    cpart_t = cpart_t + b1_ref[...]
    w2rep = jnp.concatenate([w2_ref[...]] * C, axis=0)
    rr = lax.broadcasted_iota(jnp.int32, (C * D, C), 0)
    cc = lax.broadcasted_iota(jnp.int32, (C * D, C), 1)
    w2blk = jnp.where(rr // D == cc, w2rep, 0.0)
    b2 = b2_ref[...]

    for b, db, xb in ((0, d0, x_ref[0]), (1, d1, x_ref[1])):
        ffn = db * scale + bias
        xp = lax.dot_general(w1a, ffn, (((0,), (0,)), ((), ())),
                             preferred_element_type=jnp.float32)
        h8 = jnp.concatenate(
            [jnp.maximum(xp + cpart_t[:, c:c + 1], 0.0) for c in range(C)],
            axis=0)
        sg = lax.dot_general(w2blk, h8, (((0,), (0,)), ((), ())),
                             preferred_element_type=jnp.float32)
        st = jnp.maximum(sg + b2, 0.0)
        m = jnp.max(st, axis=0, keepdims=True)
        e = jnp.exp(st - m)
        simi = e / jnp.sum(e, axis=0, keepdims=True)
        sb = lax.bitcast_convert_type(
            (simi * (1.0 / K)).astype(jnp.bfloat16), jnp.uint16)
        sw = sb.astype(jnp.uint32)
        sw = sw | (sw << 16)
        spk = lax.bitcast_convert_type(sw, jnp.float32)
        xu = lax.bitcast_convert_type(xb.astype(jnp.bfloat16),
                                      jnp.uint16).astype(jnp.uint32)
        lo = jnp.concatenate([xu[0:16], xu[32:48]], axis=0)
        hi = jnp.concatenate([xu[16:32], xu[48:64]], axis=0)
        wpk = lax.bitcast_convert_type(lo | (hi << 16), jnp.float32)
        pad = jnp.zeros((z_ref.shape[1], 128 - 32 - C), jnp.float32)
        z_ref[b] = jnp.concatenate(
            [jnp.transpose(wpk), jnp.transpose(spk), pad], axis=1)


def _stage_a(fushed_t, x_t, centroids, W1, b1c, W2, b2r, bnpack):
    grid = (-(-N // NB_A),)
    return pl.pallas_call(
        _stage_a_body,
        grid=grid,
        in_specs=[
            pl.BlockSpec((B, D, NB_A), lambda i: (0, 0, i)),
            pl.BlockSpec((B, D, NB_A), lambda i: (0, 0, i)),
            pl.BlockSpec((C, D), lambda i: (0, 0)),
            pl.BlockSpec((2 * D, D), lambda i: (0, 0)),
            pl.BlockSpec((D, 1), lambda i: (0, 0)),
            pl.BlockSpec((D, 1), lambda i: (0, 0)),
            pl.BlockSpec((1, 1), lambda i: (0, 0)),
            pl.BlockSpec((2, NB_A), lambda i: (0, i)),
        ],
        out_specs=pl.BlockSpec((B, NB_A, 128), lambda i: (0, i, 0)),
        out_shape=jax.ShapeDtypeStruct((B, N, 128), jnp.float32),
    )(fushed_t, x_t, centroids, W1, b1c, W2, b2r, bnpack)


def _stage_b_body(z_hbm, idx_hbm, out_hbm, psum_hbm,
                  idx0, idx1, idx2, zr0, zr1, zr2, out0, out1, out2,
                  psum_v, psf_v,
                  gsem0, gsem1, gsem2, isem0, isem1, isem2,
                  osem0, osem1, osem2):
    wid = lax.axis_index("s") * NC + lax.axis_index("c")
    idx_v = (idx0, idx1, idx2)
    zr = (zr0, zr1, zr2)
    out_v = (out0, out1, out2)
    gsem = (gsem0, gsem1, gsem2)
    isem = (isem0, isem1, isem2)
    osem = (osem0, osem1, osem2)
    zero_bf = jnp.zeros((32,), jnp.bfloat16)
    zero16 = jnp.zeros((16,), jnp.float32)
    for c in range(C):
        for h in range(2):
            psum_v[c, pl.ds(16 * h, 16)] = zero16

    def out_slice(cid):
        bq = cid // NCH_B
        n0 = (cid - bq * NCH_B) * CHUNK
        return out_hbm.at[bq, pl.ds(n0, CHUNK)]

    def fetch_idx(cid, p):
        bq = cid // NCH_B
        n0 = (cid - bq * NCH_B) * CHUNK
        for kk in range(K):
            pltpu.async_copy(
                idx_hbm.at[pl.ds(bq * (K * N) + kk * N + n0, CHUNK)],
                idx_v[p].at[pl.ds(kk * CHUNK, CHUNK)], isem[p])

    def finish_idx_and_gather(cid, p):
        pltpu.make_async_copy(idx_hbm.at[pl.ds(0, CHUNK * K)], idx_v[p],
                              isem[p]).wait()
        off = (cid // NCH_B) * N
        for j in range(CHUNK * K // 16):
            idx_v[p][pl.ds(16 * j, 16)] = idx_v[p][pl.ds(16 * j, 16)] + off
        pltpu.async_copy(z_hbm.at[idx_v[p]], zr[p], gsem[p])

    for p in (0, 1, 2):
        cid0 = p * NW + wid
        fetch_idx(cid0, p)
        finish_idx_and_gather(cid0, p)

    def tri_body(tt, _):
        for p in (0, 1, 2):
            q = 3 * tt + p
            cid = q * NW + wid
            cid2 = cid + 3 * NW

            @pl.when(cid < NCH)
            def _():
                pltpu.make_async_copy(z_hbm.at[idx_v[p]], zr[p],
                                      gsem[p]).wait()

                @pl.when(cid2 < NCH)
                def _():
                    fetch_idx(cid2, p)

                @pl.when(cid >= 3 * NW)
                def _():
                    pltpu.make_async_copy(out_v[p], out_slice(cid),
                                          osem[p]).wait()

                def node_body(m, psums):
                    accs = [[zero_bf, zero_bf] for _ in range(C)]
                    for k in range(K):
                        row = k * CHUNK + m
                        xh = [plsc.bitcast(zr[p][row, pl.ds(16 * h, 16)],
                                           jnp.bfloat16) for h in range(2)]
                        srow = zr[p][row, pl.ds(32, 16)]
                        for c in range(C):
                            sb = plsc.bitcast(
                                lax.broadcast_in_dim(srow[c], (16,), ()),
                                jnp.bfloat16)
                            accs[c][0] = accs[c][0] + sb * xh[0]
                            accs[c][1] = accs[c][1] + sb * xh[1]
                    new_psums = []
                    for c in range(C):
                        for h in range(2):
                            a, b = plsc.unpack(
                                accs[c][h],
                                format=plsc.PackFormat.INTERLEAVED)
                            out_v[p][m, c, pl.ds(32 * h, 16)] = a
                            out_v[p][m, c, pl.ds(32 * h + 16, 16)] = b
                            new_psums.append(psums[2 * c + h] + accs[c][h])
                    return tuple(new_psums)

                psums = lax.fori_loop(0, CHUNK, node_body,
                                      (zero_bf,) * (2 * C))
                for c in range(C):
                    for h in range(2):
                        pb = plsc.bitcast(psum_v[c, pl.ds(16 * h, 16)],
                                          jnp.bfloat16) + psums[2 * c + h]
                        psum_v[c, pl.ds(16 * h, 16)] = plsc.bitcast(
                            pb, jnp.float32)
                pltpu.async_copy(out_v[p], out_slice(cid), osem[p])

                @pl.when(cid2 < NCH)
                def _():
                    finish_idx_and_gather(cid2, p)

        return ()

    lax.fori_loop(0, QMAX, tri_body, ())
    nq = (NCH - wid + NW - 1) // NW
    for p in (0, 1, 2):
        qlast = nq - 1 - ((nq - 1 - p) % 3)
        pltpu.make_async_copy(out_v[p], out_slice(qlast * NW + wid),
                              osem[p]).wait()
    for c in range(C):
        for h in range(2):
            a, b = plsc.unpack(
                plsc.bitcast(psum_v[c, pl.ds(16 * h, 16)], jnp.bfloat16),
                format=plsc.PackFormat.INTERLEAVED)
            psf_v[c, pl.ds(32 * h, 16)] = a
            psf_v[c, pl.ds(32 * h + 16, 16)] = b
    pltpu.sync_copy(psf_v, psum_hbm.at[wid])


def _stage_b(z_rows, idx_flat):
    mesh = plsc.VectorSubcoreMesh(core_axis_name="c", subcore_axis_name="s")
    run = pl.kernel(
        _stage_b_body,
        out_type=[
            jax.ShapeDtypeStruct((B, N, C, D), jnp.float32),
            jax.ShapeDtypeStruct((NW, C, D), jnp.float32),
        ],
        mesh=mesh,
        compiler_params=pltpu.CompilerParams(needs_layout_passes=False),
        scratch_types=[
            pltpu.VMEM((CHUNK * K,), jnp.int32),
            pltpu.VMEM((CHUNK * K,), jnp.int32),
            pltpu.VMEM((CHUNK * K,), jnp.int32),
            pltpu.VMEM((CHUNK * K, 128), jnp.float32),
            pltpu.VMEM((CHUNK * K, 128), jnp.float32),
            pltpu.VMEM((CHUNK * K, 128), jnp.float32),
            pltpu.VMEM((CHUNK, C, D), jnp.float32),
            pltpu.VMEM((CHUNK, C, D), jnp.float32),
            pltpu.VMEM((CHUNK, C, D), jnp.float32),
            pltpu.VMEM((C, 32), jnp.float32),
            pltpu.VMEM((C, D), jnp.float32),
        ] + [pltpu.SemaphoreType.DMA] * 9,
    )
    return run(z_rows, idx_flat)


def _stage_c_body(psum_ref, cent_ref, out_ref):
    acc = psum_ref[0]
    for w in range(1, NW):
        acc = acc + psum_ref[w]
    u = acc * (1.0 / TOT)
    nc = (1.0 - UPDATE_RATE) * cent_ref[...] + UPDATE_RATE * u

    adj = jnp.mean(nc, axis=0, keepdims=True)
    xc = nc - adj
    nsq = jnp.sum(xc * xc, axis=1, keepdims=True)
    ones = jnp.ones_like(nsq)
    x1_ = jnp.concatenate([-2.0 * xc, nsq, ones], axis=1)
    x2_ = jnp.concatenate([xc, ones, nsq], axis=1)
    res = lax.dot_general(x1_, x2_, (((1,), (1,)), ((), ())),
                          preferred_element_type=jnp.float32)
    dist = jnp.sqrt(jnp.clip(res, 1e-30, None))
    ii = lax.broadcasted_iota(jnp.int32, (C, C), 0)
    jj = lax.broadcasted_iota(jnp.int32, (C, C), 1)
    target = jnp.where(ii == jj, 0.0, MARGIN)
    l = jnp.maximum(target - dist, 0.0)
    out_ref[...] = jnp.reshape(jnp.sum(l * l), (1, 1))


def _stage_c(psum, centroids):
    return pl.pallas_call(
        _stage_c_body,
        out_shape=jax.ShapeDtypeStruct((1, 1), jnp.float32),
    )(psum, centroids)


@jax.jit
def kernel(fushed_features, input_data, adj_mx_topk_index, centroids,
           W1, b1, W2, b2, bn_weight, bn_bias):
    b1c = jnp.reshape(b1, (D, 1))
    b2r = jnp.reshape(b2, (1, 1))
    bnpack = jnp.stack([bn_weight, bn_bias], axis=0)

    fushed_t = jnp.transpose(fushed_features, (0, 2, 1))
    x_t = jnp.reshape(jnp.transpose(input_data, (0, 1, 3, 2)), (B, D, N))

    z = _stage_a(fushed_t, x_t, centroids,
                 W1, b1c, W2, b2r, bnpack)

    idx_flat = jnp.reshape(jnp.transpose(adj_mx_topk_index, (0, 2, 1)),
                           (TOT * K,))

    z_rows = jnp.reshape(z, (TOT, 128))
    updated_input, psum = _stage_b(z_rows, idx_flat)

    loss = _stage_c(psum, centroids)
    return updated_input, jnp.reshape(loss, ())
